# Initial kernel scaffold; baseline (speedup 1.0000x reference)
import jax
import jax.numpy as jnp
from jax.experimental import pallas as pl


def kernel(coords, voxel_indices, weight, bias):
    raise NotImplementedError("write your pallas kernel here")


# speedup vs baseline: 1.7607x; 1.7607x over previous
import functools

import jax
import jax.numpy as jnp
from jax import lax
from jax.experimental import pallas as pl
from jax.experimental.pallas import tpu as pltpu
from jax.experimental.pallas import tpu_sc as plsc

D_MODEL = 128
BATCH = 16384

_info = plsc.get_sparse_core_info()
_NC, _NS = _info.num_cores, _info.num_subcores
_NW = _NC * _NS
_B_PER_W = BATCH // _NW

_mesh = plsc.VectorSubcoreMesh(core_axis_name="c", subcore_axis_name="s")


@functools.partial(
    pl.kernel,
    mesh=_mesh,
    out_type=(
        jax.ShapeDtypeStruct((BATCH, D_MODEL), jnp.float32),
        jax.ShapeDtypeStruct((BATCH,), jnp.float32),
    ),
    scratch_types=[
        pltpu.VMEM((_B_PER_W,), jnp.int32),
        pltpu.VMEM((_B_PER_W, D_MODEL), jnp.float32),
        pltpu.VMEM((_B_PER_W,), jnp.float32),
        pltpu.SemaphoreType.DMA,
        pltpu.SemaphoreType.DMA,
    ],
)
def _gather_rows(weight_hbm, bias_hbm, idx_hbm, out_w_hbm, out_b_hbm,
                 idx_v, rows_v, bvals_v, sem_w, sem_b):
    wid = lax.axis_index("s") * _NC + lax.axis_index("c")
    base = wid * _B_PER_W
    pltpu.sync_copy(idx_hbm.at[pl.ds(base, _B_PER_W)], idx_v)
    cw = pltpu.async_copy(weight_hbm.at[idx_v], rows_v, sem_w)
    cb = pltpu.async_copy(bias_hbm.at[idx_v], bvals_v, sem_b)
    cw.wait()
    pltpu.sync_copy(rows_v, out_w_hbm.at[pl.ds(base, _B_PER_W)])
    cb.wait()
    pltpu.sync_copy(bvals_v, out_b_hbm.at[pl.ds(base, _B_PER_W)])


def kernel(coords, voxel_indices, weight, bias):
    del coords
    idx = voxel_indices.astype(jnp.int32)
    return _gather_rows(weight, bias, idx)


# speedup vs baseline: 1.7677x; 1.0040x over previous
import functools

import jax
import jax.numpy as jnp
from jax import lax
from jax.experimental import pallas as pl
from jax.experimental.pallas import tpu as pltpu
from jax.experimental.pallas import tpu_sc as plsc

D_MODEL = 128
BATCH = 16384

_info = plsc.get_sparse_core_info()
_NC, _NS = _info.num_cores, _info.num_subcores
_NW = _NC * _NS
_B_PER_W = BATCH // _NW

_mesh = plsc.VectorSubcoreMesh(core_axis_name="c", subcore_axis_name="s")

_NCHUNK = 4
_CH = _B_PER_W // _NCHUNK


@functools.partial(
    pl.kernel,
    mesh=_mesh,
    out_type=(
        jax.ShapeDtypeStruct((BATCH, D_MODEL), jnp.float32),
        jax.ShapeDtypeStruct((BATCH,), jnp.float32),
    ),
    scratch_types=[
        pltpu.VMEM((_B_PER_W,), jnp.int32),
        pltpu.VMEM((_B_PER_W, D_MODEL), jnp.float32),
        pltpu.VMEM((_B_PER_W,), jnp.float32),
    ]
    + [pltpu.SemaphoreType.DMA] * (2 * _NCHUNK + 2),
)
def _gather_rows(weight_hbm, bias_hbm, idx_hbm, out_w_hbm, out_b_hbm,
                 idx_v, rows_v, bvals_v, *sems):
    wid = lax.axis_index("s") * _NC + lax.axis_index("c")
    base = wid * _B_PER_W
    pltpu.sync_copy(idx_hbm.at[pl.ds(base, _B_PER_W)], idx_v)
    cb = pltpu.async_copy(bias_hbm.at[idx_v], bvals_v, sems[2 * _NCHUNK])
    gathers = [
        pltpu.async_copy(
            weight_hbm.at[idx_v.at[pl.ds(c * _CH, _CH)]],
            rows_v.at[pl.ds(c * _CH, _CH)],
            sems[c],
        )
        for c in range(_NCHUNK)
    ]
    writes = []
    for c in range(_NCHUNK):
        gathers[c].wait()
        writes.append(
            pltpu.async_copy(
                rows_v.at[pl.ds(c * _CH, _CH)],
                out_w_hbm.at[pl.ds(base + c * _CH, _CH)],
                sems[_NCHUNK + c],
            )
        )
    cb.wait()
    writes.append(
        pltpu.async_copy(
            bvals_v, out_b_hbm.at[pl.ds(base, _B_PER_W)], sems[2 * _NCHUNK + 1]
        )
    )
    for w in writes:
        w.wait()


def kernel(coords, voxel_indices, weight, bias):
    del coords
    idx = voxel_indices.astype(jnp.int32)
    return _gather_rows(weight, bias, idx)
